# baseline (speedup 1.0000x reference)
import functools

import jax
import jax.numpy as jnp
from jax import lax
from jax.experimental import pallas as pl
from jax.experimental.pallas import tpu as pltpu
from jax.experimental.pallas import tpu_sc as plsc

NC = 2
NS = 16
LANES = 16
NW = NC * NS

BATCH = 4
MAXLEN = 8192
DIM = 768

ROWS = BATCH * MAXLEN
ROWS_PER_W = ROWS // NW
CHUNK_ROWS = 64
CHUNK_WORDS = CHUNK_ROWS * DIM
N_CHUNKS = ROWS_PER_W // CHUNK_ROWS
WORKERS_PER_BATCH = MAXLEN // ROWS_PER_W


def _sc_add(x_hbm, pos_hbm, out_hbm, xbuf, pbuf):
    wid = lax.axis_index("s") * NC + lax.axis_index("c")
    x_base = wid * ROWS_PER_W * DIM
    pos_base = (wid % WORKERS_PER_BATCH) * ROWS_PER_W * DIM

    def chunk_body(c, _):
        off = c * CHUNK_WORDS
        pltpu.sync_copy(x_hbm.at[pl.ds(x_base + off, CHUNK_WORDS)], xbuf)
        pltpu.sync_copy(pos_hbm.at[pl.ds(pos_base + off, CHUNK_WORDS)], pbuf)

        def vec_body(i, _):
            s = pl.ds(i * LANES, LANES)
            xbuf[s] = xbuf[s] + pbuf[s]
            return 0

        lax.fori_loop(0, CHUNK_WORDS // LANES, vec_body, 0)
        pltpu.sync_copy(xbuf, out_hbm.at[pl.ds(x_base + off, CHUNK_WORDS)])
        return 0

    lax.fori_loop(0, N_CHUNKS, chunk_body, 0)


_sc_kernel = pl.kernel(
    _sc_add,
    out_type=jax.ShapeDtypeStruct((ROWS * DIM,), jnp.float32),
    mesh=plsc.VectorSubcoreMesh(core_axis_name="c", subcore_axis_name="s"),
    scratch_types=[
        pltpu.VMEM((CHUNK_WORDS,), jnp.float32),
        pltpu.VMEM((CHUNK_WORDS,), jnp.float32),
    ],
)


def kernel(x, pos_table):
    out = _sc_kernel(x.reshape(-1), pos_table.reshape(-1))
    return out.reshape(x.shape)


# speedup vs baseline: 1.6541x; 1.6541x over previous
import functools

import jax
import jax.numpy as jnp
from jax import lax
from jax.experimental import pallas as pl
from jax.experimental.pallas import tpu as pltpu
from jax.experimental.pallas import tpu_sc as plsc

NC = 2
NS = 16
LANES = 16
NW = NC * NS

BATCH = 4
MAXLEN = 8192
DIM = 768

ROWS = BATCH * MAXLEN
ROWS_PER_W = ROWS // NW
CHUNK_ROWS = 32
CHUNK_WORDS = CHUNK_ROWS * DIM
N_CHUNKS = ROWS_PER_W // CHUNK_ROWS
WORKERS_PER_BATCH = MAXLEN // ROWS_PER_W
UNROLL = 8


def _sc_add(x_hbm, pos_hbm, out_hbm,
            xbuf0, xbuf1, pbuf0, pbuf1,
            sx0, sx1, sp0, sp1, so0, so1):
    wid = lax.axis_index("s") * NC + lax.axis_index("c")
    x_base = wid * ROWS_PER_W * DIM
    pos_base = (wid % WORKERS_PER_BATCH) * ROWS_PER_W * DIM

    xbufs, pbufs = (xbuf0, xbuf1), (pbuf0, pbuf1)
    sxs, sps, sos = (sx0, sx1), (sp0, sp1), (so0, so1)

    def start_load(c):
        slot = c % 2
        off = c * CHUNK_WORDS
        xh = pltpu.async_copy(
            x_hbm.at[pl.ds(x_base + off, CHUNK_WORDS)], xbufs[slot], sxs[slot])
        ph = pltpu.async_copy(
            pos_hbm.at[pl.ds(pos_base + off, CHUNK_WORDS)], pbufs[slot],
            sps[slot])
        return xh, ph

    def compute(slot):
        xb, pb = xbufs[slot], pbufs[slot]

        def vec_body(i, _):
            base = i * (LANES * UNROLL)
            for u in range(UNROLL):
                s = pl.ds(base + u * LANES, LANES)
                xb[s] = xb[s] + pb[s]
            return 0

        lax.fori_loop(0, CHUNK_WORDS // (LANES * UNROLL), vec_body, 0)

    loads = [None, None]
    stores = [None, None]
    for c in range(N_CHUNKS + 1):
        slot = c % 2
        if c < N_CHUNKS:
            if stores[slot] is not None:
                stores[slot].wait()
            loads[slot] = start_load(c)
        if c >= 1:
            pslot = (c - 1) % 2
            xh, ph = loads[pslot]
            xh.wait()
            ph.wait()
            compute(pslot)
            off = (c - 1) * CHUNK_WORDS
            stores[pslot] = pltpu.async_copy(
                xbufs[pslot], out_hbm.at[pl.ds(x_base + off, CHUNK_WORDS)],
                sos[pslot])
    stores[(N_CHUNKS - 1) % 2].wait()


_sc_kernel = pl.kernel(
    _sc_add,
    out_type=jax.ShapeDtypeStruct((ROWS * DIM,), jnp.float32),
    mesh=plsc.VectorSubcoreMesh(core_axis_name="c", subcore_axis_name="s"),
    scratch_types=[
        pltpu.VMEM((CHUNK_WORDS,), jnp.float32),
        pltpu.VMEM((CHUNK_WORDS,), jnp.float32),
        pltpu.VMEM((CHUNK_WORDS,), jnp.float32),
        pltpu.VMEM((CHUNK_WORDS,), jnp.float32),
        pltpu.SemaphoreType.DMA,
        pltpu.SemaphoreType.DMA,
        pltpu.SemaphoreType.DMA,
        pltpu.SemaphoreType.DMA,
        pltpu.SemaphoreType.DMA,
        pltpu.SemaphoreType.DMA,
    ],
)


def kernel(x, pos_table):
    out = _sc_kernel(x.reshape(-1), pos_table.reshape(-1))
    return out.reshape(x.shape)


# speedup vs baseline: 1.8230x; 1.1021x over previous
import jax
import jax.numpy as jnp
from jax import lax
from jax.experimental import pallas as pl
from jax.experimental.pallas import tpu as pltpu
from jax.experimental.pallas import tpu_sc as plsc

NC = 2
NS = 16
LANES = 16
NW = NC * NS

BATCH = 4
MAXLEN = 8192
DIM = 768

SEQ_PER_W = MAXLEN // NW
CHUNK_SEQ = 16
CHUNK_WORDS = CHUNK_SEQ * DIM
N_CHUNKS = SEQ_PER_W // CHUNK_SEQ
UNROLL = 2


def _sc_add(x_hbm, pos_hbm, out_hbm,
            x00, x01, x02, x03, x10, x11, x12, x13,
            pb0, pb1,
            sx0, sx1, sp0, sp1, so0, so1):
    wid = lax.axis_index("s") * NC + lax.axis_index("c")
    seq_base = wid * SEQ_PER_W

    xbufs = ((x00, x01, x02, x03), (x10, x11, x12, x13))
    pbufs = (pb0, pb1)
    sxs, sps, sos = (sx0, sx1), (sp0, sp1), (so0, so1)

    def x_off(b, c):
        return (b * MAXLEN + seq_base + c * CHUNK_SEQ) * DIM

    def load(c):
        slot = c % 2
        hs = [pltpu.async_copy(
            pos_hbm.at[pl.ds((seq_base + c * CHUNK_SEQ) * DIM, CHUNK_WORDS)],
            pbufs[slot], sps[slot])]
        for b in range(BATCH):
            hs.append(pltpu.async_copy(
                x_hbm.at[pl.ds(x_off(b, c), CHUNK_WORDS)],
                xbufs[slot][b], sxs[slot]))
        return hs

    def store(c):
        slot = c % 2
        return [pltpu.async_copy(
            xbufs[slot][b], out_hbm.at[pl.ds(x_off(b, c), CHUNK_WORDS)],
            sos[slot]) for b in range(BATCH)]

    def compute(slot):
        pb = pbufs[slot]
        xbs = xbufs[slot]

        def vec_body(i, _):
            base = i * (LANES * UNROLL)
            for u in range(UNROLL):
                s = pl.ds(base + u * LANES, LANES)
                p = pb[s]
                for b in range(BATCH):
                    xbs[b][s] = xbs[b][s] + p
            return 0

        lax.fori_loop(0, CHUNK_WORDS // (LANES * UNROLL), vec_body, 0)

    loads = [None, None]
    stores = [None, None]
    loads[0] = load(0)
    for c in range(N_CHUNKS):
        slot = c % 2
        if c + 1 < N_CHUNKS:
            nslot = (c + 1) % 2
            if stores[nslot] is not None:
                for h in stores[nslot]:
                    h.wait()
                stores[nslot] = None
            loads[nslot] = load(c + 1)
        for h in loads[slot]:
            h.wait()
        compute(slot)
        stores[slot] = store(c)
    for hs in stores:
        if hs is not None:
            for h in hs:
                h.wait()


_sc_kernel = pl.kernel(
    _sc_add,
    out_type=jax.ShapeDtypeStruct((BATCH * MAXLEN * DIM,), jnp.float32),
    mesh=plsc.VectorSubcoreMesh(core_axis_name="c", subcore_axis_name="s"),
    scratch_types=(
        [pltpu.VMEM((CHUNK_WORDS,), jnp.float32) for _ in range(8)]
        + [pltpu.VMEM((CHUNK_WORDS,), jnp.float32) for _ in range(2)]
        + [pltpu.SemaphoreType.DMA for _ in range(6)]
    ),
)


def kernel(x, pos_table):
    out = _sc_kernel(x.reshape(-1), pos_table.reshape(-1))
    return out.reshape(x.shape)
